# baseline (speedup 1.0000x reference)
import functools

import jax
import jax.numpy as jnp
from jax.experimental import pallas as pl
from jax.experimental.pallas import tpu as pltpu
from jax.experimental.pallas import tpu_sc as plsc

B, INPUT_DIMS, CODE_DIM, K, NUM_ACTIONS = 8192, 768, 64, 512, 768
BM = 2048


def _encode_kernel(x_ref, W1_ref, b1_ref, W2_ref, b2_ref, W3_ref, b3_ref,
                   cbT_ref, idx_ref):
    f32 = jnp.float32
    dot = functools.partial(jnp.dot, preferred_element_type=f32)

    h = jnp.maximum(dot(x_ref[:], W1_ref[:]) + b1_ref[:], 0.0)
    h = jnp.maximum(dot(h, W2_ref[:]) + b2_ref[:], 0.0)
    z_e = dot(h, W3_ref[:]) + b3_ref[:]

    cbT = cbT_ref[:]
    cnorm = jnp.sum(cbT * cbT, axis=0)[None, :]
    scores = jnp.dot(z_e, cbT, preferred_element_type=f32,
                     precision=jax.lax.Precision.HIGHEST)
    d2 = cnorm - 2.0 * scores

    dmin = jnp.min(d2, axis=1, keepdims=True)
    lane = jax.lax.broadcasted_iota(jnp.int32, d2.shape, 1)
    masked = jnp.where(d2 <= dmin, lane, K)
    idx_ref[:] = jnp.min(masked, axis=1, keepdims=True)


def _decode_kernel(zq_ref, D1_ref, d1_ref, D2_ref, d2_ref, D3_ref, d3_ref,
                   D4_ref, d4_ref, out_ref):
    f32 = jnp.float32
    dot = functools.partial(jnp.dot, preferred_element_type=f32)
    h = jnp.maximum(dot(zq_ref[:], D1_ref[:]) + d1_ref[:], 0.0)
    h = jnp.maximum(dot(h, D2_ref[:]) + d2_ref[:], 0.0)
    h = jnp.maximum(dot(h, D3_ref[:]) + d3_ref[:], 0.0)
    out_ref[:] = dot(h, D4_ref[:]) + d4_ref[:]


def _sc_gather(codebook, idx):
    GD = codebook.shape[1]
    info = plsc.get_sparse_core_info()
    NC, NS = info.num_cores, info.num_subcores
    NW = NC * NS
    b_per_w = B // NW
    mesh = plsc.VectorSubcoreMesh(core_axis_name="c", subcore_axis_name="s")

    @functools.partial(
        pl.kernel, mesh=mesh,
        out_type=jax.ShapeDtypeStruct((B, GD), jnp.float32),
        scratch_types=[
            pltpu.VMEM((b_per_w,), jnp.int32),
            pltpu.VMEM((b_per_w, GD), jnp.float32),
            pltpu.SemaphoreType.DMA,
        ],
    )
    def k(table_hbm, idx_hbm, out_hbm, idx_v, rows_v, sem):
        wid = jax.lax.axis_index("s") * NC + jax.lax.axis_index("c")
        base = wid * b_per_w
        pltpu.sync_copy(idx_hbm.at[pl.ds(base, b_per_w)], idx_v)
        pltpu.async_copy(table_hbm.at[idx_v], rows_v, sem).wait()
        pltpu.sync_copy(rows_v, out_hbm.at[pl.ds(base, b_per_w)])

    return k(codebook, idx)


def kernel(x, W1, b1, W2, b2, W3, b3, codebook, D1, d1, D2, d2, D3, d3, D4, d4):
    b1r, b2r, b3r = b1[None, :], b2[None, :], b3[None, :]
    d1r, d2r, d3r, d4r = d1[None, :], d2[None, :], d3[None, :], d4[None, :]
    cbT = codebook.T

    grid = (B // BM,)
    row_spec = lambda n: pl.BlockSpec((BM, n), lambda i: (i, 0))
    full = lambda a: pl.BlockSpec(a.shape, lambda i: tuple(0 for _ in a.shape))

    idx = pl.pallas_call(
        _encode_kernel,
        grid=grid,
        in_specs=[
            row_spec(INPUT_DIMS),
            full(W1), full(b1r), full(W2), full(b2r), full(W3), full(b3r),
            full(cbT),
        ],
        out_specs=pl.BlockSpec((BM, 1), lambda i: (i, 0)),
        out_shape=jax.ShapeDtypeStruct((B, 1), jnp.int32),
        compiler_params=pltpu.CompilerParams(
            dimension_semantics=("parallel",),
        ),
    )(x, W1, b1r, W2, b2r, W3, b3r, cbT)

    cb_pad = jnp.pad(codebook, ((0, 0), (0, 128 - CODE_DIM)))
    z_q = _sc_gather(cb_pad, idx.reshape(B))[:, :CODE_DIM]

    out = pl.pallas_call(
        _decode_kernel,
        grid=grid,
        in_specs=[
            row_spec(CODE_DIM),
            full(D1), full(d1r), full(D2), full(d2r), full(D3), full(d3r),
            full(D4), full(d4r),
        ],
        out_specs=row_spec(NUM_ACTIONS),
        out_shape=jax.ShapeDtypeStruct((B, NUM_ACTIONS), jnp.float32),
        compiler_params=pltpu.CompilerParams(
            dimension_semantics=("parallel",),
        ),
    )(z_q, D1, d1r, D2, d2r, D3, d3r, D4, d4r)
    return out


# speedup vs baseline: 8.0592x; 8.0592x over previous
import functools

import jax
import jax.numpy as jnp
from jax.experimental import pallas as pl
from jax.experimental.pallas import tpu as pltpu

B, INPUT_DIMS, CODE_DIM, K, NUM_ACTIONS = 8192, 768, 64, 512, 768
BM = 2048


def _block_kernel(x_ref, W1_ref, b1_ref, W2_ref, b2_ref, W3_ref, b3_ref,
                  cb_ref, cbT_ref, D1_ref, d1_ref, D2_ref,
                  d2_ref, D3_ref, d3_ref, D4_ref, d4_ref, out_ref):
    f32 = jnp.float32
    dot = functools.partial(jnp.dot, preferred_element_type=f32)

    x = x_ref[:]
    h = jnp.maximum(dot(x, W1_ref[:]) + b1_ref[:], 0.0)
    h = jnp.maximum(dot(h, W2_ref[:]) + b2_ref[:], 0.0)
    z_e = dot(h, W3_ref[:]) + b3_ref[:]

    cbT = cbT_ref[:]
    cnorm = jnp.sum(cbT * cbT, axis=0)[None, :]
    scores = jnp.dot(z_e, cbT, preferred_element_type=f32,
                     precision=jax.lax.Precision.HIGHEST)
    d2 = cnorm - 2.0 * scores

    dmin = jnp.min(d2, axis=1, keepdims=True)
    lane = jax.lax.broadcasted_iota(jnp.int32, d2.shape, 1)
    masked = jnp.where(d2 <= dmin, lane, K)
    idx = jnp.min(masked, axis=1, keepdims=True)

    onehot = (lane == idx).astype(f32)
    z_q = dot(onehot, cb_ref[:])

    h = jnp.maximum(dot(z_q, D1_ref[:]) + d1_ref[:], 0.0)
    h = jnp.maximum(dot(h, D2_ref[:]) + d2_ref[:], 0.0)
    h = jnp.maximum(dot(h, D3_ref[:]) + d3_ref[:], 0.0)
    out_ref[:] = dot(h, D4_ref[:]) + d4_ref[:]


def kernel(x, W1, b1, W2, b2, W3, b3, codebook, D1, d1, D2, d2, D3, d3, D4, d4):
    b1r, b2r, b3r = b1[None, :], b2[None, :], b3[None, :]
    d1r, d2r, d3r, d4r = d1[None, :], d2[None, :], d3[None, :], d4[None, :]
    cbT = codebook.T

    grid = (B // BM,)
    row_spec = lambda n: pl.BlockSpec((BM, n), lambda i: (i, 0))
    full = lambda a: pl.BlockSpec(a.shape, lambda i: tuple(0 for _ in a.shape))

    out = pl.pallas_call(
        _block_kernel,
        grid=grid,
        in_specs=[
            row_spec(INPUT_DIMS),
            full(W1), full(b1r), full(W2), full(b2r), full(W3), full(b3r),
            full(codebook), full(cbT),
            full(D1), full(d1r), full(D2), full(d2r), full(D3), full(d3r),
            full(D4), full(d4r),
        ],
        out_specs=row_spec(NUM_ACTIONS),
        out_shape=jax.ShapeDtypeStruct((B, NUM_ACTIONS), jnp.float32),
        compiler_params=pltpu.CompilerParams(
            dimension_semantics=("parallel",),
        ),
    )(x, W1, b1r, W2, b2r, W3, b3r, codebook, cbT,
      D1, d1r, D2, d2r, D3, d3r, D4, d4r)
    return out
